# Initial kernel scaffold; baseline (speedup 1.0000x reference)
import jax
import jax.numpy as jnp
from jax.experimental import pallas as pl


def kernel(x_producer, x_injector, edge_index_pi, edge_index_ip, Wp, bp, Wi, bi, Wl_pi, Wr_pi, b_pi, Wl_ip, Wr_ip, b_ip):
    raise NotImplementedError("write your pallas kernel here")


# speedup vs baseline: 2.5524x; 2.5524x over previous
import functools

import jax
import jax.numpy as jnp
from jax import lax
from jax.experimental import pallas as pl
from jax.experimental.pallas import tpu as pltpu
from jax.experimental.pallas import tpu_sc as plsc

N = 10000
E = 320000
D_IN = 128
H = 256
HH = H // 2

NC = 2
NS = 16
CHUNK = 128
RT = (-(-E // (NS * CHUNK)) + 7) // 8 * 8
ROWS_TOTAL = RT * NS
EP = ROWS_TOTAL * CHUNK
GRP = 8
ACC_ROWS = 10240
ZR = ACC_ROWS // NS
DUMMY = N

BLK = 400


def _proj_body(xp_ref, xi_ref, Wp_ref, bp_ref, Wi_ref, bi_ref,
               Wlpi_ref, Wrpi_ref, bpi_ref, Wlip_ref, Wrip_ref, bip_ref,
               gpi0_ref, gpi1_ref, gip0_ref, gip1_ref, dp_ref, di_ref):
    hp = jnp.maximum(
        jnp.dot(xp_ref[...], Wp_ref[...], preferred_element_type=jnp.float32)
        + bp_ref[...], 0.0)
    hi = jnp.maximum(
        jnp.dot(xi_ref[...], Wi_ref[...], preferred_element_type=jnp.float32)
        + bi_ref[...], 0.0)
    gpi = jnp.dot(hp, Wlpi_ref[...], preferred_element_type=jnp.float32)
    gip = jnp.dot(hi, Wlip_ref[...], preferred_element_type=jnp.float32)
    gpi0_ref[...] = gpi[:, :HH]
    gpi1_ref[...] = gpi[:, HH:]
    gip0_ref[...] = gip[:, :HH]
    gip1_ref[...] = gip[:, HH:]
    dp_ref[...] = (jnp.dot(hp, Wrip_ref[...], preferred_element_type=jnp.float32)
                   + bip_ref[...])
    di_ref[...] = (jnp.dot(hi, Wrpi_ref[...], preferred_element_type=jnp.float32)
                   + bpi_ref[...])


def _tc_proj(xp, xi, Wp, bp, Wi, bi, Wl_pi, Wr_pi, b_pi, Wl_ip, Wr_ip, b_ip):
    grid = (N // BLK,)
    row_blk = lambda w: pl.BlockSpec((BLK, w), lambda i: (i, 0))
    full = lambda a, b: pl.BlockSpec((a, b), lambda i: (0, 0))
    return pl.pallas_call(
        _proj_body,
        grid=grid,
        in_specs=[
            row_blk(D_IN), row_blk(D_IN),
            full(D_IN, H), full(1, H), full(D_IN, H), full(1, H),
            full(H, H), full(H, H), full(1, H),
            full(H, H), full(H, H), full(1, H),
        ],
        out_specs=[row_blk(HH)] * 4 + [row_blk(H), row_blk(H)],
        out_shape=[jax.ShapeDtypeStruct((N, HH), jnp.float32)] * 4
        + [jax.ShapeDtypeStruct((N, H), jnp.float32)] * 2,
    )(xp, xi, Wp, bp.reshape(1, H), Wi, bi.reshape(1, H),
      Wl_pi, Wr_pi, b_pi.reshape(1, H), Wl_ip, Wr_ip, b_ip.reshape(1, H))


def _sc_body(gpi0, gpi1, gip0, gip1, src_pi, dst_pi, src_ip, dst_ip,
             Spi0, Spi1, Sip0, Sip1, cnt_pi, cnt_ip,
             src_v, dst_v, rows_v, acc, sem):
    c = lax.axis_index("c")
    s = lax.axis_index("s")

    def fill_rows(val):
        def _row(i, _):
            for g in range(HH // 16):
                rows_v[i, pl.ds(g * 16, 16)] = jnp.full((16,), val, jnp.float32)
            return 0
        lax.fori_loop(0, CHUNK, _row, 0)

    def run_relation(src2d, dst2d, g_halves, out_tabs):
        for p in range(3):
            fill_rows(0.0)
            for z in range(ZR // CHUNK):
                pltpu.sync_copy(rows_v, acc.at[pl.ds(s * ZR + z * CHUNK, CHUNK)])
            plsc.subcore_barrier()
            if p == 2:
                fill_rows(1.0)

            def group_body(gi, _):
                off = pl.multiple_of(s * RT + gi * GRP, GRP)
                pltpu.sync_copy(dst2d.at[pl.ds(off, GRP)], dst_v)
                if p < 2:
                    pltpu.sync_copy(src2d.at[pl.ds(off, GRP)], src_v)
                for j in range(GRP):
                    if p < 2:
                        pltpu.async_copy(
                            g_halves[p].at[src_v.at[j]], rows_v, sem).wait()
                    pltpu.sync_copy(rows_v, acc.at[dst_v.at[j]], add=True)
                return 0
            lax.fori_loop(0, RT // GRP, group_body, 0)
            plsc.subcore_barrier()

            nch = jnp.where(s == NS - 1, 5, 8)
            bounce = rows_v.at[pl.ds(0, 80)]

            def dump_body(k, _):
                off = s * ZR + k * 80
                pltpu.sync_copy(acc.at[pl.ds(off, 80)], bounce)
                pltpu.sync_copy(bounce, out_tabs[p].at[pl.ds(off, 80)])
                return 0
            lax.fori_loop(0, nch, dump_body, 0)
            plsc.subcore_barrier()

    @pl.when(c == 0)
    def _():
        run_relation(src_pi, dst_pi, (gpi0, gpi1), (Spi0, Spi1, cnt_pi))

    @pl.when(c == 1)
    def _():
        run_relation(src_ip, dst_ip, (gip0, gip1), (Sip0, Sip1, cnt_ip))


_sc_agg = functools.partial(
    pl.kernel,
    out_type=[jax.ShapeDtypeStruct((N, HH), jnp.float32)] * 4
    + [jax.ShapeDtypeStruct((N, HH), jnp.float32)] * 2,
    mesh=plsc.VectorSubcoreMesh(
        core_axis_name="c", subcore_axis_name="s",
        num_cores=NC, num_subcores=NS),
    scratch_types=[
        pltpu.VMEM((GRP, CHUNK), jnp.int32),
        pltpu.VMEM((GRP, CHUNK), jnp.int32),
        pltpu.VMEM((CHUNK, HH), jnp.float32),
        pltpu.VMEM_SHARED((ACC_ROWS, HH), jnp.float32),
        pltpu.SemaphoreType.DMA,
    ],
)(_sc_body)


def _final_body(Sip0, Sip1, cip, dp, Spi0, Spi1, cpi, di, out_ref):
    inv_p = 1.0 / jnp.maximum(cip[:, :1], 1.0)
    inv_i = 1.0 / jnp.maximum(cpi[:, :1], 1.0)
    mean_p = jnp.concatenate([Sip0[...], Sip1[...]], axis=1) * inv_p
    mean_i = jnp.concatenate([Spi0[...], Spi1[...]], axis=1) * inv_i
    out_ref[0] = jnp.maximum(mean_p + dp[...], 0.0)
    out_ref[1] = jnp.maximum(mean_i + di[...], 0.0)


def _tc_final(Sip0, Sip1, cnt_ip, d_p, Spi0, Spi1, cnt_pi, d_i):
    grid = (N // BLK,)
    row_blk = lambda w: pl.BlockSpec((BLK, w), lambda i: (i, 0))
    return pl.pallas_call(
        _final_body,
        grid=grid,
        in_specs=[row_blk(HH), row_blk(HH), row_blk(HH), row_blk(H),
                  row_blk(HH), row_blk(HH), row_blk(HH), row_blk(H)],
        out_specs=pl.BlockSpec((2, BLK, H), lambda i: (0, i, 0)),
        out_shape=jax.ShapeDtypeStruct((2, N, H), jnp.float32),
    )(Sip0, Sip1, cnt_ip, d_p, Spi0, Spi1, cnt_pi, d_i)


def _pad_edges(edge_index):
    pad = EP - E
    src = jnp.concatenate(
        [edge_index[0], jnp.zeros((pad,), jnp.int32)]).reshape(ROWS_TOTAL, CHUNK)
    dst = jnp.concatenate(
        [edge_index[1], jnp.full((pad,), DUMMY, jnp.int32)]).reshape(ROWS_TOTAL, CHUNK)
    return src, dst


def kernel(x_producer, x_injector, edge_index_pi, edge_index_ip,
           Wp, bp, Wi, bi, Wl_pi, Wr_pi, b_pi, Wl_ip, Wr_ip, b_ip):
    gpi0, gpi1, gip0, gip1, d_p, d_i = _tc_proj(
        x_producer, x_injector, Wp, bp, Wi, bi,
        Wl_pi, Wr_pi, b_pi, Wl_ip, Wr_ip, b_ip)
    src_pi, dst_pi = _pad_edges(edge_index_pi)
    src_ip, dst_ip = _pad_edges(edge_index_ip)
    Spi0, Spi1, Sip0, Sip1, cnt_pi, cnt_ip = _sc_agg(
        gpi0, gpi1, gip0, gip1, src_pi, dst_pi, src_ip, dst_ip)
    return _tc_final(Sip0, Sip1, cnt_ip, d_p, Spi0, Spi1, cnt_pi, d_i)


# speedup vs baseline: 2.6331x; 1.0316x over previous
import functools

import jax
import jax.numpy as jnp
from jax import lax
from jax.experimental import pallas as pl
from jax.experimental.pallas import tpu as pltpu
from jax.experimental.pallas import tpu_sc as plsc

N = 10000
E = 320000
D_IN = 128
H = 256
HH = H // 2

NC = 2
NS = 16
CHUNK = 64
GRP = 8
RT = (-(-E // (NS * CHUNK)) + GRP - 1) // GRP * GRP
ROWS_TOTAL = RT * NS
EP = ROWS_TOTAL * CHUNK
NGRP = RT // GRP
ACC_ROWS = 10240
ZR = ACC_ROWS // NS
DUMMY = N

BLK = 400


def _proj_body(xp_ref, xi_ref, Wp_ref, bp_ref, Wi_ref, bi_ref,
               Wlpi_ref, Wrpi_ref, bpi_ref, Wlip_ref, Wrip_ref, bip_ref,
               gpi0_ref, gpi1_ref, gip0_ref, gip1_ref, dp_ref, di_ref):
    hp = jnp.maximum(
        jnp.dot(xp_ref[...], Wp_ref[...], preferred_element_type=jnp.float32)
        + bp_ref[...], 0.0)
    hi = jnp.maximum(
        jnp.dot(xi_ref[...], Wi_ref[...], preferred_element_type=jnp.float32)
        + bi_ref[...], 0.0)
    gpi = jnp.dot(hp, Wlpi_ref[...], preferred_element_type=jnp.float32)
    gip = jnp.dot(hi, Wlip_ref[...], preferred_element_type=jnp.float32)
    gpi0_ref[...] = gpi[:, :HH]
    gpi1_ref[...] = gpi[:, HH:]
    gip0_ref[...] = gip[:, :HH]
    gip1_ref[...] = gip[:, HH:]
    dp_ref[...] = (jnp.dot(hp, Wrip_ref[...], preferred_element_type=jnp.float32)
                   + bip_ref[...])
    di_ref[...] = (jnp.dot(hi, Wrpi_ref[...], preferred_element_type=jnp.float32)
                   + bpi_ref[...])


def _tc_proj(xp, xi, Wp, bp, Wi, bi, Wl_pi, Wr_pi, b_pi, Wl_ip, Wr_ip, b_ip):
    grid = (N // BLK,)
    row_blk = lambda w: pl.BlockSpec((BLK, w), lambda i: (i, 0))
    full = lambda a, b: pl.BlockSpec((a, b), lambda i: (0, 0))
    return pl.pallas_call(
        _proj_body,
        grid=grid,
        in_specs=[
            row_blk(D_IN), row_blk(D_IN),
            full(D_IN, H), full(1, H), full(D_IN, H), full(1, H),
            full(H, H), full(H, H), full(1, H),
            full(H, H), full(H, H), full(1, H),
        ],
        out_specs=[row_blk(HH)] * 4 + [row_blk(H), row_blk(H)],
        out_shape=[jax.ShapeDtypeStruct((N, HH), jnp.float32)] * 4
        + [jax.ShapeDtypeStruct((N, H), jnp.float32)] * 2,
    )(xp, xi, Wp, bp.reshape(1, H), Wi, bi.reshape(1, H),
      Wl_pi, Wr_pi, b_pi.reshape(1, H), Wl_ip, Wr_ip, b_ip.reshape(1, H))


def _sc_body(gpi0, gpi1, gip0, gip1, src_pi, dst_pi, src_ip, dst_ip,
             Spi0, Spi1, Sip0, Sip1, cnt_pi, cnt_ip,
             src_v, dst_v, rows_a, rows_b, acc, sem, sem2):
    c = lax.axis_index("c")
    s = lax.axis_index("s")
    bufs = (rows_a, rows_b)

    def fill_rows(buf, val):
        def _row(i, _):
            for g in range(HH // 16):
                buf[i, pl.ds(g * 16, 16)] = jnp.full((16,), val, jnp.float32)
            return 0
        lax.fori_loop(0, CHUNK, _row, 0)

    def run_relation(src2d, dst2d, g_halves, out_tabs):
        for p in range(3):
            fill_rows(rows_a, 0.0)
            for z in range(ZR // CHUNK):
                pltpu.sync_copy(rows_a, acc.at[pl.ds(s * ZR + z * CHUNK, CHUNK)])
            plsc.subcore_barrier()
            if p == 2:
                fill_rows(rows_a, 1.0)

            if p < 2:
                g = g_halves[p]

                def group_body(gi, _):
                    off = pl.multiple_of(s * RT + gi * GRP, GRP)
                    pltpu.sync_copy(dst2d.at[pl.ds(off, GRP)], dst_v)
                    pltpu.sync_copy(src2d.at[pl.ds(off, GRP)], src_v)
                    cp = pltpu.async_copy(g.at[src_v.at[0]], bufs[0], sem)
                    for k in range(GRP):
                        cp.wait()
                        if k + 1 < GRP:
                            cp = pltpu.async_copy(
                                g.at[src_v.at[k + 1]], bufs[(k + 1) % 2], sem)
                        pltpu.sync_copy(
                            bufs[k % 2], acc.at[dst_v.at[k]], add=True)
                    return 0
            else:
                def group_body(gi, _):
                    off = pl.multiple_of(s * RT + gi * GRP, GRP)
                    pltpu.sync_copy(dst2d.at[pl.ds(off, GRP)], dst_v)
                    descs = [
                        pltpu.async_copy(
                            rows_a, acc.at[dst_v.at[k]], sem2, add=True)
                        for k in range(GRP)
                    ]
                    for d in descs:
                        d.wait()
                    return 0
            lax.fori_loop(0, NGRP, group_body, 0)
            plsc.subcore_barrier()

            nch = jnp.where(s == NS - 1, 10, 16)
            bounce = rows_a.at[pl.ds(0, 40)]

            def dump_body(k, _):
                off = s * ZR + k * 40
                pltpu.sync_copy(acc.at[pl.ds(off, 40)], bounce)
                pltpu.sync_copy(bounce, out_tabs[p].at[pl.ds(off, 40)])
                return 0
            lax.fori_loop(0, nch, dump_body, 0)
            plsc.subcore_barrier()

    @pl.when(c == 0)
    def _():
        run_relation(src_pi, dst_pi, (gpi0, gpi1), (Spi0, Spi1, cnt_pi))

    @pl.when(c == 1)
    def _():
        run_relation(src_ip, dst_ip, (gip0, gip1), (Sip0, Sip1, cnt_ip))


_sc_agg = functools.partial(
    pl.kernel,
    out_type=[jax.ShapeDtypeStruct((N, HH), jnp.float32)] * 4
    + [jax.ShapeDtypeStruct((N, HH), jnp.float32)] * 2,
    mesh=plsc.VectorSubcoreMesh(
        core_axis_name="c", subcore_axis_name="s",
        num_cores=NC, num_subcores=NS),
    scratch_types=[
        pltpu.VMEM((GRP, CHUNK), jnp.int32),
        pltpu.VMEM((GRP, CHUNK), jnp.int32),
        pltpu.VMEM((CHUNK, HH), jnp.float32),
        pltpu.VMEM((CHUNK, HH), jnp.float32),
        pltpu.VMEM_SHARED((ACC_ROWS, HH), jnp.float32),
        pltpu.SemaphoreType.DMA,
        pltpu.SemaphoreType.DMA,
    ],
)(_sc_body)


def _final_body(Sip0, Sip1, cip, dp, Spi0, Spi1, cpi, di, out_ref):
    inv_p = 1.0 / jnp.maximum(cip[:, :1], 1.0)
    inv_i = 1.0 / jnp.maximum(cpi[:, :1], 1.0)
    mean_p = jnp.concatenate([Sip0[...], Sip1[...]], axis=1) * inv_p
    mean_i = jnp.concatenate([Spi0[...], Spi1[...]], axis=1) * inv_i
    out_ref[0] = jnp.maximum(mean_p + dp[...], 0.0)
    out_ref[1] = jnp.maximum(mean_i + di[...], 0.0)


def _tc_final(Sip0, Sip1, cnt_ip, d_p, Spi0, Spi1, cnt_pi, d_i):
    grid = (N // BLK,)
    row_blk = lambda w: pl.BlockSpec((BLK, w), lambda i: (i, 0))
    return pl.pallas_call(
        _final_body,
        grid=grid,
        in_specs=[row_blk(HH), row_blk(HH), row_blk(HH), row_blk(H),
                  row_blk(HH), row_blk(HH), row_blk(HH), row_blk(H)],
        out_specs=pl.BlockSpec((2, BLK, H), lambda i: (0, i, 0)),
        out_shape=jax.ShapeDtypeStruct((2, N, H), jnp.float32),
    )(Sip0, Sip1, cnt_ip, d_p, Spi0, Spi1, cnt_pi, d_i)


def _pad_edges(edge_index):
    pad = EP - E
    src = jnp.concatenate(
        [edge_index[0], jnp.zeros((pad,), jnp.int32)]).reshape(ROWS_TOTAL, CHUNK)
    dst = jnp.concatenate(
        [edge_index[1], jnp.full((pad,), DUMMY, jnp.int32)]).reshape(ROWS_TOTAL, CHUNK)
    return src, dst


def kernel(x_producer, x_injector, edge_index_pi, edge_index_ip,
           Wp, bp, Wi, bi, Wl_pi, Wr_pi, b_pi, Wl_ip, Wr_ip, b_ip):
    gpi0, gpi1, gip0, gip1, d_p, d_i = _tc_proj(
        x_producer, x_injector, Wp, bp, Wi, bi,
        Wl_pi, Wr_pi, b_pi, Wl_ip, Wr_ip, b_ip)
    src_pi, dst_pi = _pad_edges(edge_index_pi)
    src_ip, dst_ip = _pad_edges(edge_index_ip)
    Spi0, Spi1, Sip0, Sip1, cnt_pi, cnt_ip = _sc_agg(
        gpi0, gpi1, gip0, gip1, src_pi, dst_pi, src_ip, dst_ip)
    return _tc_final(Sip0, Sip1, cnt_ip, d_p, Spi0, Spi1, cnt_pi, d_i)


# speedup vs baseline: 3.7378x; 1.4195x over previous
import functools

import jax
import jax.numpy as jnp
from jax import lax
from jax.experimental import pallas as pl
from jax.experimental.pallas import tpu as pltpu
from jax.experimental.pallas import tpu_sc as plsc

N = 10000
E = 320000
D_IN = 128
H = 256
HH = H // 2

NC = 2
NS = 16
CHUNK = 40
NB = 4
GRP = 8
RT = (-(-E // (NS * CHUNK)) + GRP - 1) // GRP * GRP
ROWS_TOTAL = RT * NS
EP = ROWS_TOTAL * CHUNK
NGRP = RT // GRP
ACC_ROWS = 10240
ZR = ACC_ROWS // NS
DUMMY = N
CSLOTS = 10240

BLK = 400


def _proj_body(xp_ref, xi_ref, Wp_ref, bp_ref, Wi_ref, bi_ref,
               Wlpi_ref, Wrpi_ref, bpi_ref, Wlip_ref, Wrip_ref, bip_ref,
               gpi0_ref, gpi1_ref, gip0_ref, gip1_ref, dp_ref, di_ref):
    hp = jnp.maximum(
        jnp.dot(xp_ref[...], Wp_ref[...], preferred_element_type=jnp.float32)
        + bp_ref[...], 0.0)
    hi = jnp.maximum(
        jnp.dot(xi_ref[...], Wi_ref[...], preferred_element_type=jnp.float32)
        + bi_ref[...], 0.0)
    gpi = jnp.dot(hp, Wlpi_ref[...], preferred_element_type=jnp.float32)
    gip = jnp.dot(hi, Wlip_ref[...], preferred_element_type=jnp.float32)
    gpi0_ref[...] = gpi[:, :HH]
    gpi1_ref[...] = gpi[:, HH:]
    gip0_ref[...] = gip[:, :HH]
    gip1_ref[...] = gip[:, HH:]
    dp_ref[...] = (jnp.dot(hp, Wrip_ref[...], preferred_element_type=jnp.float32)
                   + bip_ref[...])
    di_ref[...] = (jnp.dot(hi, Wrpi_ref[...], preferred_element_type=jnp.float32)
                   + bpi_ref[...])


def _tc_proj(xp, xi, Wp, bp, Wi, bi, Wl_pi, Wr_pi, b_pi, Wl_ip, Wr_ip, b_ip):
    grid = (N // BLK,)
    row_blk = lambda w: pl.BlockSpec((BLK, w), lambda i: (i, 0))
    full = lambda a, b: pl.BlockSpec((a, b), lambda i: (0, 0))
    return pl.pallas_call(
        _proj_body,
        grid=grid,
        in_specs=[
            row_blk(D_IN), row_blk(D_IN),
            full(D_IN, H), full(1, H), full(D_IN, H), full(1, H),
            full(H, H), full(H, H), full(1, H),
            full(H, H), full(H, H), full(1, H),
        ],
        out_specs=[row_blk(HH)] * 4 + [row_blk(H), row_blk(H)],
        out_shape=[jax.ShapeDtypeStruct((N, HH), jnp.float32)] * 4
        + [jax.ShapeDtypeStruct((N, H), jnp.float32)] * 2,
    )(xp, xi, Wp, bp.reshape(1, H), Wi, bi.reshape(1, H),
      Wl_pi, Wr_pi, b_pi.reshape(1, H), Wl_ip, Wr_ip, b_ip.reshape(1, H))


def _sc_body(gpi0, gpi1, gip0, gip1, src_pi, dst_pi, src_ip, dst_ip,
             Spi0, Spi1, Sip0, Sip1, cnt_pi, cnt_ip,
             src_v, dst_v, b0, b1, b2, b3, acc,
             g0, g1, g2, g3, s0, s1, s2, s3):
    c = lax.axis_index("c")
    s = lax.axis_index("s")
    bufs = (b0, b1, b2, b3)
    gsem = (g0, g1, g2, g3)
    ssem = (s0, s1, s2, s3)

    def fill_rows(buf, nrows, val):
        def _row(i, _):
            for g in range(HH // 16):
                buf[i, pl.ds(g * 16, 16)] = jnp.full((16,), val, jnp.float32)
            return 0
        lax.fori_loop(0, nrows, _row, 0)

    def run_relation(src2d, dst2d, g_halves, out_tabs):
        for p in range(3):
            fill_rows(b0, CHUNK, 0.0)
            for z in range(ZR // CHUNK):
                pltpu.sync_copy(b0, acc.at[pl.ds(s * ZR + z * CHUNK, CHUNK)])
            plsc.subcore_barrier()
            if p == 2:
                fill_rows(b0, CHUNK, 1.0)

            if p < 2:
                g = g_halves[p]

                def group_body(gi, _):
                    off = pl.multiple_of(s * RT + gi * GRP, GRP)
                    pltpu.sync_copy(dst2d.at[pl.ds(off, GRP)], dst_v)
                    pltpu.sync_copy(src2d.at[pl.ds(off, GRP)], src_v)
                    gd = [None] * GRP
                    sd = [None] * GRP
                    for j in range(NB - 1):
                        gd[j] = pltpu.async_copy(
                            g.at[src_v.at[j]], bufs[j % NB], gsem[j % NB])
                    for k in range(GRP):
                        b = k % NB
                        gd[k].wait()
                        sd[k] = pltpu.async_copy(
                            bufs[b], acc.at[dst_v.at[k]], ssem[b], add=True)
                        j = k + NB - 1
                        if j < GRP:
                            bj = j % NB
                            if j >= NB:
                                sd[j - NB].wait()
                            gd[j] = pltpu.async_copy(
                                g.at[src_v.at[j]], bufs[bj], gsem[bj])
                    for k in range(GRP - NB, GRP):
                        sd[k].wait()
                    return 0
            else:
                def group_body(gi, _):
                    off = pl.multiple_of(s * RT + gi * GRP, GRP)
                    pltpu.sync_copy(dst2d.at[pl.ds(off, GRP)], dst_v)
                    descs = [
                        pltpu.async_copy(
                            b0, acc.at[dst_v.at[k]], s0, add=True)
                        for k in range(GRP)
                    ]
                    for d in descs:
                        d.wait()
                    return 0
            lax.fori_loop(0, NGRP, group_body, 0)
            plsc.subcore_barrier()

            nch = jnp.where(s == NS - 1, 25, 40)
            bounce = b0.at[pl.ds(0, 16)]

            def dump_body(k, _):
                off = s * ZR + k * 16
                pltpu.sync_copy(acc.at[pl.ds(off, 16)], bounce)
                pltpu.sync_copy(bounce, out_tabs[p].at[pl.ds(off, 16)])
                return 0
            lax.fori_loop(0, nch, dump_body, 0)
            plsc.subcore_barrier()

    @pl.when(c == 0)
    def _():
        run_relation(src_pi, dst_pi, (gpi0, gpi1), (Spi0, Spi1, cnt_pi))

    @pl.when(c == 1)
    def _():
        run_relation(src_ip, dst_ip, (gip0, gip1), (Sip0, Sip1, cnt_ip))


_sc_agg = functools.partial(
    pl.kernel,
    out_type=[jax.ShapeDtypeStruct((N, HH), jnp.float32)] * 4
    + [jax.ShapeDtypeStruct((N, HH), jnp.float32)] * 2,
    mesh=plsc.VectorSubcoreMesh(
        core_axis_name="c", subcore_axis_name="s",
        num_cores=NC, num_subcores=NS),
    scratch_types=[
        pltpu.VMEM((GRP, CHUNK), jnp.int32),
        pltpu.VMEM((GRP, CHUNK), jnp.int32),
        pltpu.VMEM((CHUNK, HH), jnp.float32),
        pltpu.VMEM((CHUNK, HH), jnp.float32),
        pltpu.VMEM((CHUNK, HH), jnp.float32),
        pltpu.VMEM((CHUNK, HH), jnp.float32),
        pltpu.VMEM_SHARED((ACC_ROWS, HH), jnp.float32),
    ] + [pltpu.SemaphoreType.DMA] * 8,
)(_sc_body)


def _final_body(Sip0, Sip1, cip, dp, Spi0, Spi1, cpi, di, out_ref):
    inv_p = 1.0 / jnp.maximum(cip[:, :1], 1.0)
    inv_i = 1.0 / jnp.maximum(cpi[:, :1], 1.0)
    mean_p = jnp.concatenate([Sip0[...], Sip1[...]], axis=1) * inv_p
    mean_i = jnp.concatenate([Spi0[...], Spi1[...]], axis=1) * inv_i
    out_ref[0] = jnp.maximum(mean_p + dp[...], 0.0)
    out_ref[1] = jnp.maximum(mean_i + di[...], 0.0)


def _tc_final(Sip0, Sip1, cnt_ip, d_p, Spi0, Spi1, cnt_pi, d_i):
    grid = (N // BLK,)
    row_blk = lambda w: pl.BlockSpec((BLK, w), lambda i: (i, 0))
    return pl.pallas_call(
        _final_body,
        grid=grid,
        in_specs=[row_blk(HH), row_blk(HH), row_blk(HH), row_blk(H),
                  row_blk(HH), row_blk(HH), row_blk(HH), row_blk(H)],
        out_specs=pl.BlockSpec((2, BLK, H), lambda i: (0, i, 0)),
        out_shape=jax.ShapeDtypeStruct((2, N, H), jnp.float32),
    )(Sip0, Sip1, cnt_ip, d_p, Spi0, Spi1, cnt_pi, d_i)


def _pad_edges(edge_index):
    pad = EP - E
    src = jnp.concatenate(
        [edge_index[0], jnp.zeros((pad,), jnp.int32)]).reshape(ROWS_TOTAL, CHUNK)
    dst = jnp.concatenate(
        [edge_index[1], jnp.full((pad,), DUMMY, jnp.int32)]).reshape(ROWS_TOTAL, CHUNK)
    return src, dst


def kernel(x_producer, x_injector, edge_index_pi, edge_index_ip,
           Wp, bp, Wi, bi, Wl_pi, Wr_pi, b_pi, Wl_ip, Wr_ip, b_ip):
    gpi0, gpi1, gip0, gip1, d_p, d_i = _tc_proj(
        x_producer, x_injector, Wp, bp, Wi, bi,
        Wl_pi, Wr_pi, b_pi, Wl_ip, Wr_ip, b_ip)
    src_pi, dst_pi = _pad_edges(edge_index_pi)
    src_ip, dst_ip = _pad_edges(edge_index_ip)
    Spi0, Spi1, Sip0, Sip1, cnt_pi, cnt_ip = _sc_agg(
        gpi0, gpi1, gip0, gip1, src_pi, dst_pi, src_ip, dst_ip)
    return _tc_final(Sip0, Sip1, cnt_ip, d_p, Spi0, Spi1, cnt_pi, d_i)
